# baseline (speedup 1.0000x reference)
import jax
import jax.numpy as jnp
from jax.experimental import pallas as pl
from jax.experimental.pallas import tpu as pltpu
from jax.experimental.pallas import tpu_sc as plsc

_TOP_K = 8

_ROT = ((13, 15, 26, 6), (17, 29, 16, 24))


def _gauss_block(flat_u32):
    k0 = jnp.uint32(0)
    k1 = jnp.uint32(42)
    ks = (k0, k1, k0 ^ k1 ^ jnp.uint32(0x1BD11BDA))
    x0 = jnp.zeros_like(flat_u32) + ks[0]
    x1 = flat_u32 + ks[1]
    for i in range(5):
        for r in _ROT[i % 2]:
            x0 = x0 + x1
            x1 = (x1 << jnp.uint32(r)) | (x1 >> jnp.uint32(32 - r))
            x1 = x1 ^ x0
        x0 = x0 + ks[(i + 1) % 3]
        x1 = x1 + ks[(i + 2) % 3] + jnp.uint32(i + 1)
    bits = x0 ^ x1

    u = pltpu.bitcast((bits >> jnp.uint32(9)) | jnp.uint32(0x3F800000),
                      jnp.float32) - 1.0
    lo = jnp.float32(-0.99999994)
    x = u * (jnp.float32(1.0) - lo) + lo
    w = -jnp.log1p(-x * x)
    small = w < 5.0
    ws = w - 2.5
    wl = jnp.sqrt(w) - 3.0
    p1 = jnp.full_like(x, 2.81022636e-08)
    for c in (3.43273939e-07, -3.5233877e-06, -4.39150654e-06, 0.00021858087,
              -0.00125372503, -0.00417768164, 0.246640727, 1.50140941):
        p1 = p1 * ws + jnp.float32(c)
    p2 = jnp.full_like(x, -0.000200214257)
    for c in (0.000100950558, 0.00134934322, -0.00367342844, 0.00573950773,
              -0.0076224613, 0.00943887047, 1.00167406, 2.83297682):
        p2 = p2 * wl + jnp.float32(c)
    p = jnp.where(small, p1, p2)
    return jnp.float32(1.4142135623730951) * (p * x)


def _router_block_kernel(x_ref, wcat_ref, bcat_ref, out_ref, idx_ref, acc_ref):
    rows, twoexp = acc_ref.shape
    n_exp = twoexp // 2

    acc = acc_ref[...]
    logits = acc[:, :n_exp]
    nlog = acc[:, n_exp:]

    g_rows, g_cols = rows // 2, twoexp
    row_i = jax.lax.broadcasted_iota(jnp.int32, (g_rows, g_cols), 0)
    col_i = jax.lax.broadcasted_iota(jnp.int32, (g_rows, g_cols), 1)
    base = (pl.program_id(0) - 1) * (rows * n_exp)
    flat = (base + row_i * n_exp + col_i
            + jnp.where(col_i >= n_exp, (g_rows - 1) * n_exp, 0)
            ).astype(jnp.uint32)
    g2 = _gauss_block(flat)
    gauss = jnp.concatenate([g2[:, :n_exp], g2[:, n_exp:]], axis=0)

    noisy = logits + gauss * jax.nn.softplus(nlog)

    chunk = 64 if rows % 64 == 0 else rows
    neg_inf = jnp.float32(-jnp.inf)
    for c in range(rows // chunk):
        nz = noisy[c * chunk:(c + 1) * chunk, :]
        colf = jax.lax.broadcasted_iota(jnp.int32, nz.shape, 1).astype(jnp.float32)
        kcol = jax.lax.broadcasted_iota(jnp.int32, (chunk, _TOP_K), 1)
        idx_out = jnp.zeros((chunk, _TOP_K), jnp.int32)
        work = nz
        m0 = None
        for k in range(_TOP_K):
            m = jnp.max(work, axis=1, keepdims=True)
            amaxf = jnp.min(jnp.where(work == m, colf, jnp.float32(128.0)),
                            axis=1, keepdims=True)
            idx_out = jnp.where(kcol == k, amaxf.astype(jnp.int32), idx_out)
            if k == 0:
                m0 = m
            work = jnp.where(colf == amaxf, neg_inf, work)
        idx_ref[c * chunk:(c + 1) * chunk, :] = idx_out

        mask = work == neg_inf
        e = jnp.where(mask, jnp.exp(nz - m0), 0.0)
        r = jnp.float32(1.0) / jnp.sum(e, axis=1, keepdims=True)
        out_ref[c * chunk:(c + 1) * chunk, :] = e * r

    acc_ref[...] = jnp.dot(x_ref[...], wcat_ref[...],
                           preferred_element_type=jnp.float32) + bcat_ref[...]


def kernel(x, W_route, b_route, W_noise, b_noise):
    n_tokens, d_model = x.shape
    n_experts = W_route.shape[0]
    blk = 512 if n_tokens % 512 == 0 else n_tokens
    nblk = n_tokens // blk
    grid = (nblk + 1,)

    wcat = jnp.concatenate([W_route.T, W_noise.T], axis=1)
    bcat = jnp.concatenate([b_route, b_noise]).reshape(1, 2 * n_experts)

    router, indices = pl.pallas_call(
        _router_block_kernel,
        grid=grid,
        in_specs=[
            pl.BlockSpec((blk, d_model), lambda i: (jnp.minimum(i, nblk - 1), 0)),
            pl.BlockSpec((d_model, 2 * n_experts), lambda i: (0, 0)),
            pl.BlockSpec((1, 2 * n_experts), lambda i: (0, 0)),
        ],
        out_specs=[
            pl.BlockSpec((blk, n_experts), lambda i: (jnp.maximum(i - 1, 0), 0)),
            pl.BlockSpec((blk, _TOP_K), lambda i: (jnp.maximum(i - 1, 0), 0)),
        ],
        out_shape=[
            jax.ShapeDtypeStruct((n_tokens, n_experts), jnp.float32),
            jax.ShapeDtypeStruct((n_tokens, _TOP_K), jnp.int32),
        ],
        scratch_shapes=[pltpu.VMEM((blk, 2 * n_experts), jnp.float32)],
        compiler_params=pltpu.CompilerParams(
            dimension_semantics=("arbitrary",),
        ),
    )(x, wcat, bcat)

    def _sc_copy_body(inp_ref, out_ref, vbuf, sem_in, sem_out):
        c = jax.lax.axis_index("c")
        sub = jax.lax.axis_index("s") + 16 * c
        rows = inp_ref.shape[0] // 32
        start = sub * rows
        cp = pltpu.make_async_copy(inp_ref.at[pl.ds(start, rows), :], vbuf, sem_in)
        cp.start()
        cp.wait()
        cp2 = pltpu.make_async_copy(vbuf, out_ref.at[pl.ds(start, rows), :], sem_out)
        cp2.start()
        cp2.wait()

    indices = pl.kernel(
        _sc_copy_body,
        out_type=jax.ShapeDtypeStruct((n_tokens, _TOP_K), jnp.int32),
        mesh=plsc.VectorSubcoreMesh(core_axis_name="c", subcore_axis_name="s"),
        scratch_types=[
            pltpu.MemorySpace.VMEM(( n_tokens // 32, _TOP_K), jnp.int32),
            pltpu.SemaphoreType.DMA,
            pltpu.SemaphoreType.DMA,
        ],
    )(indices)
    return (router, indices)


# speedup vs baseline: 1.1420x; 1.1420x over previous
import jax
import jax.numpy as jnp
from jax.experimental import pallas as pl
from jax.experimental.pallas import tpu as pltpu

_TOP_K = 8

_ROT = ((13, 15, 26, 6), (17, 29, 16, 24))


def _gauss_block(flat_u32):
    k0 = jnp.uint32(0)
    k1 = jnp.uint32(42)
    ks = (k0, k1, k0 ^ k1 ^ jnp.uint32(0x1BD11BDA))
    x0 = jnp.zeros_like(flat_u32) + ks[0]
    x1 = flat_u32 + ks[1]
    for i in range(5):
        for r in _ROT[i % 2]:
            x0 = x0 + x1
            x1 = (x1 << jnp.uint32(r)) | (x1 >> jnp.uint32(32 - r))
            x1 = x1 ^ x0
        x0 = x0 + ks[(i + 1) % 3]
        x1 = x1 + ks[(i + 2) % 3] + jnp.uint32(i + 1)
    bits = x0 ^ x1

    u = pltpu.bitcast((bits >> jnp.uint32(9)) | jnp.uint32(0x3F800000),
                      jnp.float32) - 1.0
    lo = jnp.float32(-0.99999994)
    x = u * (jnp.float32(1.0) - lo) + lo
    w = -jnp.log1p(-x * x)
    small = w < 5.0
    ws = w - 2.5
    wl = jnp.sqrt(w) - 3.0
    p1 = jnp.full_like(x, 2.81022636e-08)
    for c in (3.43273939e-07, -3.5233877e-06, -4.39150654e-06, 0.00021858087,
              -0.00125372503, -0.00417768164, 0.246640727, 1.50140941):
        p1 = p1 * ws + jnp.float32(c)
    p2 = jnp.full_like(x, -0.000200214257)
    for c in (0.000100950558, 0.00134934322, -0.00367342844, 0.00573950773,
              -0.0076224613, 0.00943887047, 1.00167406, 2.83297682):
        p2 = p2 * wl + jnp.float32(c)
    p = jnp.where(small, p1, p2)
    return jnp.float32(1.4142135623730951) * (p * x)


def _router_block_kernel(x_ref, wcat_ref, bcat_ref, out_ref, idx_ref, acc_ref):
    rows, twoexp = acc_ref.shape
    n_exp = twoexp // 2

    acc = acc_ref[...]
    logits = acc[:, :n_exp]
    nlog = acc[:, n_exp:]

    g_rows, g_cols = rows // 2, twoexp
    row_i = jax.lax.broadcasted_iota(jnp.int32, (g_rows, g_cols), 0)
    col_i = jax.lax.broadcasted_iota(jnp.int32, (g_rows, g_cols), 1)
    base = (pl.program_id(0) - 1) * (rows * n_exp)
    flat = (base + row_i * n_exp + col_i
            + jnp.where(col_i >= n_exp, (g_rows - 1) * n_exp, 0)
            ).astype(jnp.uint32)
    g2 = _gauss_block(flat)
    gauss = jnp.concatenate([g2[:, :n_exp], g2[:, n_exp:]], axis=0)

    noisy = logits + gauss * jax.nn.softplus(nlog)

    chunk = 64 if rows % 64 == 0 else rows
    neg_inf = jnp.float32(-jnp.inf)
    for c in range(rows // chunk):
        nz = noisy[c * chunk:(c + 1) * chunk, :]
        colf = jax.lax.broadcasted_iota(jnp.int32, nz.shape, 1).astype(jnp.float32)
        kcol = jax.lax.broadcasted_iota(jnp.int32, (chunk, _TOP_K), 1)
        idx_out = jnp.zeros((chunk, _TOP_K), jnp.int32)
        work = nz
        m0 = None
        for k in range(_TOP_K):
            m = jnp.max(work, axis=1, keepdims=True)
            amaxf = jnp.min(jnp.where(work == m, colf, jnp.float32(128.0)),
                            axis=1, keepdims=True)
            idx_out = jnp.where(kcol == k, amaxf.astype(jnp.int32), idx_out)
            if k == 0:
                m0 = m
            work = jnp.where(colf == amaxf, neg_inf, work)
        idx_ref[c * chunk:(c + 1) * chunk, :] = idx_out

        mask = work == neg_inf
        e = jnp.where(mask, jnp.exp(nz - m0), 0.0)
        r = jnp.float32(1.0) / jnp.sum(e, axis=1, keepdims=True)
        out_ref[c * chunk:(c + 1) * chunk, :] = e * r

    acc_ref[...] = jnp.dot(x_ref[...], wcat_ref[...],
                           preferred_element_type=jnp.float32) + bcat_ref[...]


def kernel(x, W_route, b_route, W_noise, b_noise):
    n_tokens, d_model = x.shape
    n_experts = W_route.shape[0]
    blk = 512 if n_tokens % 512 == 0 else n_tokens
    nblk = n_tokens // blk
    grid = (nblk + 1,)

    wcat = jnp.concatenate([W_route.T, W_noise.T], axis=1)
    bcat = jnp.concatenate([b_route, b_noise]).reshape(1, 2 * n_experts)

    router, indices = pl.pallas_call(
        _router_block_kernel,
        grid=grid,
        in_specs=[
            pl.BlockSpec((blk, d_model), lambda i: (jnp.minimum(i, nblk - 1), 0)),
            pl.BlockSpec((d_model, 2 * n_experts), lambda i: (0, 0)),
            pl.BlockSpec((1, 2 * n_experts), lambda i: (0, 0)),
        ],
        out_specs=[
            pl.BlockSpec((blk, n_experts), lambda i: (jnp.maximum(i - 1, 0), 0)),
            pl.BlockSpec((blk, _TOP_K), lambda i: (jnp.maximum(i - 1, 0), 0)),
        ],
        out_shape=[
            jax.ShapeDtypeStruct((n_tokens, n_experts), jnp.float32),
            jax.ShapeDtypeStruct((n_tokens, _TOP_K), jnp.int32),
        ],
        scratch_shapes=[pltpu.VMEM((blk, 2 * n_experts), jnp.float32)],
        compiler_params=pltpu.CompilerParams(
            dimension_semantics=("parallel",),
        ),
    )(x, wcat, bcat)
    return (router, indices)


# speedup vs baseline: 1.1437x; 1.0014x over previous
import jax
import jax.numpy as jnp
from jax.experimental import pallas as pl
from jax.experimental.pallas import tpu as pltpu

_TOP_K = 8

_ROT = ((13, 15, 26, 6), (17, 29, 16, 24))


def _gauss_block(flat_u32):
    k0 = jnp.uint32(0)
    k1 = jnp.uint32(42)
    ks = (k0, k1, k0 ^ k1 ^ jnp.uint32(0x1BD11BDA))
    x0 = jnp.zeros_like(flat_u32) + ks[0]
    x1 = flat_u32 + ks[1]
    for i in range(5):
        for r in _ROT[i % 2]:
            x0 = x0 + x1
            x1 = (x1 << jnp.uint32(r)) | (x1 >> jnp.uint32(32 - r))
            x1 = x1 ^ x0
        x0 = x0 + ks[(i + 1) % 3]
        x1 = x1 + ks[(i + 2) % 3] + jnp.uint32(i + 1)
    bits = x0 ^ x1

    u = pltpu.bitcast((bits >> jnp.uint32(9)) | jnp.uint32(0x3F800000),
                      jnp.float32) - 1.0
    lo = jnp.float32(-0.99999994)
    x = u * (jnp.float32(1.0) - lo) + lo
    w = -jnp.log1p(-x * x)
    small = w < 5.0
    ws = w - 2.5
    wl = jnp.sqrt(w) - 3.0
    p1 = jnp.full_like(x, 2.81022636e-08)
    for c in (3.43273939e-07, -3.5233877e-06, -4.39150654e-06, 0.00021858087,
              -0.00125372503, -0.00417768164, 0.246640727, 1.50140941):
        p1 = p1 * ws + jnp.float32(c)
    p2 = jnp.full_like(x, -0.000200214257)
    for c in (0.000100950558, 0.00134934322, -0.00367342844, 0.00573950773,
              -0.0076224613, 0.00943887047, 1.00167406, 2.83297682):
        p2 = p2 * wl + jnp.float32(c)
    p = jnp.where(small, p1, p2)
    return jnp.float32(1.4142135623730951) * (p * x)


def _router_block_kernel(x_ref, wcat_ref, bcat_ref, flat0_ref, out_ref,
                         idx_ref, acc_ref):
    rows, twoexp = acc_ref.shape
    n_exp = twoexp // 2

    acc = acc_ref[...]
    logits = acc[:, :n_exp]
    nlog = acc[:, n_exp:]

    base = (pl.program_id(0) - 1) * (rows * n_exp)
    flat = flat0_ref[...] + jnp.uint32(base)
    g2 = _gauss_block(flat)
    gauss = jnp.concatenate([g2[:, :n_exp], g2[:, n_exp:]], axis=0)

    noisy = logits + gauss * jax.nn.softplus(nlog)

    chunk = 64 if rows % 64 == 0 else rows
    neg_inf = jnp.float32(-jnp.inf)
    for c in range(rows // chunk):
        nz = noisy[c * chunk:(c + 1) * chunk, :]
        colf = jax.lax.broadcasted_iota(jnp.int32, nz.shape, 1).astype(jnp.float32)
        kcol = jax.lax.broadcasted_iota(jnp.int32, (chunk, _TOP_K), 1)
        idx_out = jnp.zeros((chunk, _TOP_K), jnp.int32)
        work = nz
        m0 = None
        for k in range(_TOP_K):
            m = jnp.max(work, axis=1, keepdims=True)
            amaxf = jnp.min(jnp.where(work == m, colf, jnp.float32(128.0)),
                            axis=1, keepdims=True)
            idx_out = jnp.where(kcol == k, amaxf.astype(jnp.int32), idx_out)
            if k == 0:
                m0 = m
            work = jnp.where(colf == amaxf, neg_inf, work)
        idx_ref[c * chunk:(c + 1) * chunk, :] = idx_out

        mask = work == neg_inf
        e = jnp.where(mask, jnp.exp(nz - m0), 0.0)
        r = jnp.float32(1.0) / jnp.sum(e, axis=1, keepdims=True)
        out_ref[c * chunk:(c + 1) * chunk, :] = e * r

    acc_ref[...] = jnp.dot(x_ref[...], wcat_ref[...],
                           preferred_element_type=jnp.float32) + bcat_ref[...]


def kernel(x, W_route, b_route, W_noise, b_noise):
    n_tokens, d_model = x.shape
    n_experts = W_route.shape[0]
    blk = 512 if n_tokens % 512 == 0 else n_tokens
    nblk = n_tokens // blk
    grid = (nblk + 1,)

    wcat = jnp.concatenate([W_route.T, W_noise.T], axis=1)
    bcat = jnp.concatenate([b_route, b_noise]).reshape(1, 2 * n_experts)

    g_rows = blk // 2
    row_i = jnp.arange(g_rows, dtype=jnp.int32)[:, None]
    col_i = jnp.arange(2 * n_experts, dtype=jnp.int32)[None, :]
    flat0 = (row_i * n_experts + col_i
             + jnp.where(col_i >= n_experts, (g_rows - 1) * n_experts, 0)
             ).astype(jnp.uint32)

    router, indices = pl.pallas_call(
        _router_block_kernel,
        grid=grid,
        in_specs=[
            pl.BlockSpec((blk, d_model), lambda i: (jnp.minimum(i, nblk - 1), 0)),
            pl.BlockSpec((d_model, 2 * n_experts), lambda i: (0, 0)),
            pl.BlockSpec((1, 2 * n_experts), lambda i: (0, 0)),
            pl.BlockSpec((blk // 2, 2 * n_experts), lambda i: (0, 0)),
        ],
        out_specs=[
            pl.BlockSpec((blk, n_experts), lambda i: (jnp.maximum(i - 1, 0), 0)),
            pl.BlockSpec((blk, _TOP_K), lambda i: (jnp.maximum(i - 1, 0), 0)),
        ],
        out_shape=[
            jax.ShapeDtypeStruct((n_tokens, n_experts), jnp.float32),
            jax.ShapeDtypeStruct((n_tokens, _TOP_K), jnp.int32),
        ],
        scratch_shapes=[pltpu.VMEM((blk, 2 * n_experts), jnp.float32)],
        compiler_params=pltpu.CompilerParams(
            dimension_semantics=("arbitrary",),
        ),
    )(x, wcat, bcat, flat0)
    return (router, indices)
